# Initial kernel scaffold; baseline (speedup 1.0000x reference)
import jax
import jax.numpy as jnp
from jax.experimental import pallas as pl


def kernel(table, centroids, indices):
    raise NotImplementedError("write your pallas kernel here")


# speedup vs baseline: 1.6202x; 1.6202x over previous
import functools

import jax
import jax.numpy as jnp
from jax import lax
from jax.experimental import pallas as pl
from jax.experimental.pallas import tpu as pltpu
from jax.experimental.pallas import tpu_sc as plsc

_EPS = 1e-3


def _sc_gather(table, idxs):
    _, emb = table.shape
    batch = idxs.shape[0]
    info = plsc.get_sparse_core_info()
    num_workers = info.num_cores * info.num_subcores
    bpw = batch // num_workers
    mesh = plsc.VectorSubcoreMesh(core_axis_name="c", subcore_axis_name="s")

    @functools.partial(
        pl.kernel,
        mesh=mesh,
        out_type=jax.ShapeDtypeStruct((batch, emb), jnp.float32),
        scratch_types=[
            pltpu.VMEM((bpw,), jnp.int32),
            pltpu.VMEM((bpw, emb), jnp.float32),
            pltpu.SemaphoreType.DMA,
        ],
    )
    def gather_k(table_hbm, idx_hbm, out_hbm, idx_v, rows_v, sem):
        wid = lax.axis_index("s") * info.num_cores + lax.axis_index("c")
        base = wid * bpw
        pltpu.sync_copy(idx_hbm.at[pl.ds(base, bpw)], idx_v)
        pltpu.async_copy(table_hbm.at[idx_v], rows_v, sem).wait()
        pltpu.sync_copy(rows_v, out_hbm.at[pl.ds(base, bpw)])

    return gather_k(table, idxs)


def _responses(x_ref, cents_ref, d, sub):
    z = x_ref[:, d * sub:(d + 1) * sub]
    cd = cents_ref[d]
    n1 = jnp.sum(z * z, axis=1, keepdims=True)
    n2 = jnp.sum(cd * cd, axis=1)
    dt = lax.dot_general(z, cd, (((1,), (1,)), ((), ())),
                         precision=lax.Precision.DEFAULT)
    return -n1 + 2.0 * dt - n2[None, :], z


def _stats_body(idx_ref, x_ref, cents_ref, st_ref, *, bn, cutoff, nd, sub):
    i = pl.program_id(0)
    w = (idx_ref[...] >= cutoff).astype(jnp.float32)
    rs = [_responses(x_ref, cents_ref, d, sub)[0] for d in range(nd)]

    @pl.when(i == 0)
    def _init():
        tot = rs[0]
        for r in rs[1:]:
            tot = tot + r
        c0 = jnp.sum(tot, axis=0, keepdims=True) / (float(nd) * bn)
        st_ref[...] = jnp.zeros_like(st_ref)
        st_ref[0:1, :] = c0

    c = st_ref[0:1, :]
    wt = 1.0 - w
    s1h = s2h = s1t = s2t = None
    for r in rs:
        rc = r - c
        rch = rc * w
        rct = rc * wt
        a = jnp.sum(rch, axis=0, keepdims=True)
        b = jnp.sum(rc * rch, axis=0, keepdims=True)
        e = jnp.sum(rct, axis=0, keepdims=True)
        f = jnp.sum(rc * rct, axis=0, keepdims=True)
        s1h = a if s1h is None else s1h + a
        s2h = b if s2h is None else s2h + b
        s1t = e if s1t is None else s1t + e
        s2t = f if s2t is None else s2t + f
    st_ref[1:2, :] += s1h
    st_ref[2:3, :] += s2h
    st_ref[3:4, :] += s1t
    st_ref[4:5, :] += s2t
    st_ref[5:6, :] += jnp.sum(w)


def _quant_body(idx_ref, x_ref, cents_ref, st_ref, out_ref,
                *, n, cutoff, nd, sub, kk):
    kt = kk // 4
    st = st_ref[...]
    c = st[0:1, :]
    cnth = st[5:6, :]
    denh = cnth * float(nd)
    dent = (float(n) - cnth) * float(nd)
    mh_c = st[1:2, :] / denh
    varh = st[2:3, :] / denh - mh_c * mh_c
    sh = jnp.sqrt(varh + _EPS)
    mh = c + mh_c
    mt_c = st[3:4, :] / dent
    vart = st[4:5, :] / dent - mt_c * mt_c
    stt = jnp.sqrt(vart + _EPS)
    mt = c + mt_c

    head = idx_ref[...] >= cutoff
    bn = x_ref.shape[0]
    iota = lax.broadcasted_iota(jnp.int32, (bn, kk), 1)
    for d in range(nd):
        r, z = _responses(x_ref, cents_ref, d, sub)
        rh = (r - mh) / sh
        code_h = jnp.argmax(rh, axis=1).astype(jnp.int32)
        rt = (r[:, :kt] - mt[:, :kt]) / stt[:, :kt]
        code_t = jnp.argmax(rt, axis=1).astype(jnp.int32)
        code = jnp.where(head, code_h[:, None], code_t[:, None])
        onehot = (iota == code).astype(jnp.float32)
        od = lax.dot_general(onehot, cents_ref[d],
                             (((1,), (0,)), ((), ())),
                             precision=lax.Precision.HIGHEST)
        out_ref[:, d * sub:(d + 1) * sub] = (od - z) + z


def kernel(table, centroids, indices):
    vocab, emb = table.shape
    nd, kk, sub = centroids.shape
    cutoff = int(vocab * 0.8)
    idxs = indices.reshape(-1)
    n = idxs.shape[0]

    x = _sc_gather(table, idxs)
    idx2 = idxs[:, None]

    bn = 2048
    grid = (n // bn,)
    st = pl.pallas_call(
        functools.partial(_stats_body, bn=bn, cutoff=cutoff, nd=nd, sub=sub),
        grid=grid,
        in_specs=[
            pl.BlockSpec((bn, 1), lambda i: (i, 0)),
            pl.BlockSpec((bn, emb), lambda i: (i, 0)),
            pl.BlockSpec((nd, kk, sub), lambda i: (0, 0, 0)),
        ],
        out_specs=pl.BlockSpec((8, kk), lambda i: (0, 0)),
        out_shape=jax.ShapeDtypeStruct((8, kk), jnp.float32),
    )(idx2, x, centroids)

    out = pl.pallas_call(
        functools.partial(_quant_body, n=n, cutoff=cutoff, nd=nd, sub=sub,
                          kk=kk),
        grid=grid,
        in_specs=[
            pl.BlockSpec((bn, 1), lambda i: (i, 0)),
            pl.BlockSpec((bn, emb), lambda i: (i, 0)),
            pl.BlockSpec((nd, kk, sub), lambda i: (0, 0, 0)),
            pl.BlockSpec((8, kk), lambda i: (0, 0)),
        ],
        out_specs=pl.BlockSpec((bn, emb), lambda i: (i, 0)),
        out_shape=jax.ShapeDtypeStruct((n, emb), jnp.float32),
    )(idx2, x, centroids, st)

    return out.reshape(indices.shape + (emb,))


# speedup vs baseline: 1.9996x; 1.2342x over previous
import functools

import jax
import jax.numpy as jnp
from jax import lax
from jax.experimental import pallas as pl
from jax.experimental.pallas import tpu as pltpu
from jax.experimental.pallas import tpu_sc as plsc

_EPS = 1e-3


def _sc_gather(table, idxs):
    _, emb = table.shape
    batch = idxs.shape[0]
    info = plsc.get_sparse_core_info()
    num_workers = info.num_cores * info.num_subcores
    bpw = batch // num_workers
    mesh = plsc.VectorSubcoreMesh(core_axis_name="c", subcore_axis_name="s")

    @functools.partial(
        pl.kernel,
        mesh=mesh,
        out_type=jax.ShapeDtypeStruct((batch, emb), jnp.float32),
        scratch_types=[
            pltpu.VMEM((bpw,), jnp.int32),
            pltpu.VMEM((bpw, emb), jnp.float32),
            pltpu.SemaphoreType.DMA,
        ],
    )
    def gather_k(table_hbm, idx_hbm, out_hbm, idx_v, rows_v, sem):
        wid = lax.axis_index("s") * info.num_cores + lax.axis_index("c")
        base = wid * bpw
        pltpu.sync_copy(idx_hbm.at[pl.ds(base, bpw)], idx_v)
        pltpu.async_copy(table_hbm.at[idx_v], rows_v, sem).wait()
        pltpu.sync_copy(rows_v, out_hbm.at[pl.ds(base, bpw)])

    return gather_k(table, idxs)


def _fused_body(idx_ref, x_ref, cents_ref, out_ref,
                *, n, nchunks, bn, cutoff, nd, sub, kk):
    kt = kk // 4
    f32 = jnp.float32

    n2s = []
    for d in range(nd):
        cd = cents_ref[d]
        n2s.append(jnp.sum(cd * cd, axis=1))

    def resp(base, d):
        z = x_ref[pl.ds(base, bn), d * sub:(d + 1) * sub]
        n1 = jnp.sum(z * z, axis=1, keepdims=True)
        dt = lax.dot_general(z, cents_ref[d], (((1,), (1,)), ((), ())),
                             precision=lax.Precision.DEFAULT)
        return -n1 + 2.0 * dt - n2s[d][None, :], z

    tot = resp(0, 0)[0]
    for d in range(1, nd):
        tot = tot + resp(0, d)[0]
    c = jnp.sum(tot, axis=0, keepdims=True) / (float(nd) * bn)

    def p1(i, carry):
        s1, s2, s1h, s2h, cnth = carry
        base = i * bn
        w = (idx_ref[pl.ds(base, bn), :] >= cutoff).astype(f32)
        for d in range(nd):
            rc = resp(base, d)[0] - c
            rc2 = rc * rc
            s1 = s1 + jnp.sum(rc, axis=0, keepdims=True)
            s2 = s2 + jnp.sum(rc2, axis=0, keepdims=True)
            s1h = s1h + jnp.sum(rc * w, axis=0, keepdims=True)
            s2h = s2h + jnp.sum(rc2 * w, axis=0, keepdims=True)
        cnth = cnth + jnp.sum(w)
        return (s1, s2, s1h, s2h, cnth)

    zrow = jnp.zeros((1, kk), f32)
    s1, s2, s1h, s2h, cnth = lax.fori_loop(
        0, nchunks, p1, (zrow, zrow, zrow, zrow, f32(0.0)))

    denh = cnth * float(nd)
    dent = (float(n) - cnth) * float(nd)
    mh_c = s1h / denh
    sh = jnp.sqrt(s2h / denh - mh_c * mh_c + _EPS)
    mh = c + mh_c
    invh = 1.0 / sh
    mt_c = (s1 - s1h) / dent
    stt = jnp.sqrt((s2 - s2h) / dent - mt_c * mt_c + _EPS)
    mt = (c + mt_c)[:, :kt]
    invt = (1.0 / stt)[:, :kt]

    chi = [cents_ref[d].astype(jnp.bfloat16) for d in range(nd)]
    clo = [(cents_ref[d] - chi[d].astype(f32)).astype(jnp.bfloat16)
           for d in range(nd)]

    iota = lax.broadcasted_iota(jnp.int32, (bn, kk), 1)

    def p2(i, _):
        base = i * bn
        head = idx_ref[pl.ds(base, bn), :] >= cutoff
        for d in range(nd):
            r, z = resp(base, d)
            rh = (r - mh) * invh
            code_h = jnp.argmax(rh, axis=1).astype(jnp.int32)
            rt = (r[:, :kt] - mt) * invt
            code_t = jnp.argmax(rt, axis=1).astype(jnp.int32)
            code = jnp.where(head, code_h[:, None], code_t[:, None])
            oh = (iota == code).astype(jnp.bfloat16)
            od = (lax.dot_general(oh, chi[d], (((1,), (0,)), ((), ())),
                                  preferred_element_type=f32)
                  + lax.dot_general(oh, clo[d], (((1,), (0,)), ((), ())),
                                    preferred_element_type=f32))
            out_ref[pl.ds(base, bn), d * sub:(d + 1) * sub] = (od - z) + z
        return 0

    lax.fori_loop(0, nchunks, p2, 0)


def kernel(table, centroids, indices):
    vocab, emb = table.shape
    nd, kk, sub = centroids.shape
    cutoff = int(vocab * 0.8)
    idxs = indices.reshape(-1)
    n = idxs.shape[0]

    x = _sc_gather(table, idxs)
    idx2 = idxs[:, None]

    bn = 2048
    nchunks = n // bn
    out = pl.pallas_call(
        functools.partial(_fused_body, n=n, nchunks=nchunks, bn=bn,
                          cutoff=cutoff, nd=nd, sub=sub, kk=kk),
        out_shape=jax.ShapeDtypeStruct((n, emb), jnp.float32),
    )(idx2, x, centroids)

    return out.reshape(indices.shape + (emb,))
